# baseline (speedup 1.0000x reference)
import functools

import jax
import jax.numpy as jnp
from jax import lax
from jax.experimental import pallas as pl
from jax.experimental.pallas import tpu as pltpu
from jax.experimental.pallas import tpu_sc as plsc

_N_NODES = 100
_N_STATES = 1000
_BATCH = 16384
_TSTRIDE = 1024
_TWORDS = _N_NODES * _TSTRIDE

_NW = 32
_SPW = _BATCH // _NW
_HC = 128
_NH = _SPW // _HC
_GRP = _HC // 16


def _prep_body(l_ref, tab_ref):
    l = l_ref[...]
    m = jnp.max(l, axis=1, keepdims=True)
    s = jnp.sum(jnp.exp(l - m), axis=1, keepdims=True)
    lse = jnp.log(s) + m
    t = (l - lse) * jnp.float32(1.0 / _N_NODES)
    tp = jnp.concatenate(
        [t, jnp.zeros((_N_NODES, _TSTRIDE - _N_STATES), jnp.float32)], axis=1)
    tab_ref[...] = tp.reshape(_TWORDS // 128, 128)


def _prep_table(logits):
    return pl.pallas_call(
        _prep_body,
        out_shape=jax.ShapeDtypeStruct((_TWORDS // 128, 128), jnp.float32),
    )(logits)


def _sc_gather_sum(xt, tab):
    mesh = plsc.VectorSubcoreMesh(core_axis_name="c", subcore_axis_name="s")

    @functools.partial(
        pl.kernel,
        mesh=mesh,
        out_type=jax.ShapeDtypeStruct((_BATCH,), jnp.float32),
        compiler_params=pltpu.CompilerParams(
            needs_layout_passes=False,
            disable_bounds_checks=True,
            disable_semaphore_checks=True,
            skip_device_barrier=True,
        ),
        scratch_types=[
            pltpu.VMEM((_TWORDS,), jnp.float32),
            pltpu.VMEM((_N_NODES, _HC), jnp.int32),
            pltpu.VMEM((_N_NODES, _HC), jnp.int32),
            pltpu.VMEM((_HC,), jnp.float32),
            pltpu.SemaphoreType.DMA,
            pltpu.SemaphoreType.DMA,
            pltpu.SemaphoreType.DMA,
            pltpu.SemaphoreType.DMA,
            pltpu.SemaphoreType.DMA,
        ],
    )
    def k(xt_hbm, tab_hbm, out_hbm,
          tab_v, xa_v, xb_v, out_v,
          sem_t, sem_t2, sem_xa, sem_xb, sem_o):
        wid = lax.axis_index("s") * 2 + lax.axis_index("c")
        base = wid * _SPW

        half = _TWORDS // 2
        h_t0 = pltpu.async_copy(
            tab_hbm.at[pl.ds(0, half)], tab_v.at[pl.ds(0, half)], sem_t)
        h_t1 = pltpu.async_copy(
            tab_hbm.at[pl.ds(half, half)], tab_v.at[pl.ds(half, half)], sem_t2)
        xbufs = (xa_v, xb_v)
        xsems = (sem_xa, sem_xb)
        h = [None, None]
        h[0] = pltpu.async_copy(xt_hbm.at[:, pl.ds(base, _HC)], xa_v, sem_xa)
        h[1] = pltpu.async_copy(
            xt_hbm.at[:, pl.ds(base + _HC, _HC)], xb_v, sem_xb)
        h_t0.wait()
        zero = jnp.zeros((16,), jnp.float32)

        def make_body(xv):
            def body(n, accs):
                noff = n * _TSTRIDE
                new = []
                for g in range(_GRP):
                    xrow = xv[n, pl.ds(g * 16, 16)]
                    val = plsc.load_gather(tab_v, [xrow + noff])
                    new.append(accs[g] + val)
                return tuple(new)
            return body

        for hc in range(_NH):
            h[hc % 2].wait()
            xv = xbufs[hc % 2]
            body = make_body(xv)
            if hc == 0:
                accs = lax.fori_loop(0, _N_NODES // 2, body,
                                     (zero,) * _GRP, unroll=4)
                h_t1.wait()
                accs = lax.fori_loop(_N_NODES // 2, _N_NODES, body,
                                     accs, unroll=4)
            else:
                accs = lax.fori_loop(0, _N_NODES, body,
                                     (zero,) * _GRP, unroll=4)
            if hc + 2 < _NH:
                h[hc % 2] = pltpu.async_copy(
                    xt_hbm.at[:, pl.ds(base + (hc + 2) * _HC, _HC)],
                    xbufs[hc % 2], xsems[hc % 2])
            for g in range(_GRP):
                out_v[pl.ds(g * 16, 16)] = accs[g]
            pltpu.async_copy(
                out_v, out_hbm.at[pl.ds(base + hc * _HC, _HC)], sem_o).wait()

    return k(xt, tab)


def kernel(x, logits):
    tab = _prep_table(logits)
    return _sc_gather_sum(x.T, tab.reshape(-1))


# speedup vs baseline: 1.0304x; 1.0304x over previous
import functools

import jax
import jax.numpy as jnp
from jax import lax
from jax.experimental import pallas as pl
from jax.experimental.pallas import tpu as pltpu
from jax.experimental.pallas import tpu_sc as plsc

_N_NODES = 100
_N_STATES = 1000
_BATCH = 16384
_TSTRIDE = 1024
_TWORDS = _N_NODES * _TSTRIDE

_NW = 32
_SPW = _BATCH // _NW
_HC = 128
_NH = _SPW // _HC
_GRP = _HC // 16


def _prep_body(l_ref, tab_ref):
    l = l_ref[...]
    m = jnp.max(l, axis=1, keepdims=True)
    s = jnp.sum(jnp.exp(l - m), axis=1, keepdims=True)
    lse = jnp.log(s) + m
    t = (l - lse) * jnp.float32(1.0 / _N_NODES)
    tp = jnp.concatenate(
        [t, jnp.zeros((_N_NODES, _TSTRIDE - _N_STATES), jnp.float32)], axis=1)
    tab_ref[...] = tp.reshape(_TWORDS // 128, 128)


def _prep_table(logits):
    return pl.pallas_call(
        _prep_body,
        out_shape=jax.ShapeDtypeStruct((_TWORDS // 128, 128), jnp.float32),
    )(logits)


def _sc_gather_sum(xt, tab):
    mesh = plsc.VectorSubcoreMesh(core_axis_name="c", subcore_axis_name="s")

    @functools.partial(
        pl.kernel,
        mesh=mesh,
        out_type=jax.ShapeDtypeStruct((_BATCH,), jnp.float32),
        compiler_params=pltpu.CompilerParams(needs_layout_passes=False),
        scratch_types=[
            pltpu.VMEM((_TWORDS,), jnp.float32),
            pltpu.VMEM((_N_NODES, _HC), jnp.int32),
            pltpu.VMEM((_N_NODES, _HC), jnp.int32),
            pltpu.VMEM((_SPW,), jnp.float32),
            pltpu.SemaphoreType.DMA,
            pltpu.SemaphoreType.DMA,
            pltpu.SemaphoreType.DMA,
            pltpu.SemaphoreType.DMA,
        ],
    )
    def k(xt_hbm, tab_hbm, out_hbm,
          tab_v, xa_v, xb_v, out_v,
          sem_t, sem_xa, sem_xb, sem_o):
        wid = lax.axis_index("s") * 2 + lax.axis_index("c")
        base = wid * _SPW

        h_t = pltpu.async_copy(tab_hbm, tab_v, sem_t)
        xbufs = (xa_v, xb_v)
        xsems = (sem_xa, sem_xb)
        h = [None, None]
        h[0] = pltpu.async_copy(xt_hbm.at[:, pl.ds(base, _HC)], xa_v, sem_xa)
        h[1] = pltpu.async_copy(
            xt_hbm.at[:, pl.ds(base + _HC, _HC)], xb_v, sem_xb)
        h_t.wait()
        zero = jnp.zeros((16,), jnp.float32)

        def make_body(xv):
            def body(n, accs):
                noff = n * _TSTRIDE
                new = []
                for g in range(_GRP):
                    xrow = xv[n, pl.ds(g * 16, 16)]
                    val = plsc.load_gather(tab_v, [xrow + noff])
                    new.append(accs[g] + val)
                return tuple(new)
            return body

        out_handles = []
        for hc in range(_NH):
            h[hc % 2].wait()
            xv = xbufs[hc % 2]
            accs = lax.fori_loop(0, _N_NODES, make_body(xv),
                                 (zero,) * _GRP, unroll=4)
            if hc + 2 < _NH:
                h[hc % 2] = pltpu.async_copy(
                    xt_hbm.at[:, pl.ds(base + (hc + 2) * _HC, _HC)],
                    xbufs[hc % 2], xsems[hc % 2])
            for g in range(_GRP):
                out_v[pl.ds(hc * _HC + g * 16, 16)] = accs[g]
            out_handles.append(pltpu.async_copy(
                out_v.at[pl.ds(hc * _HC, _HC)],
                out_hbm.at[pl.ds(base + hc * _HC, _HC)], sem_o))
        for oh in out_handles:
            oh.wait()

    return k(xt, tab)


def kernel(x, logits):
    tab = _prep_table(logits)
    return _sc_gather_sum(x.T, tab.reshape(-1))


# speedup vs baseline: 1.0380x; 1.0074x over previous
import functools

import jax
import jax.numpy as jnp
from jax import lax
from jax.experimental import pallas as pl
from jax.experimental.pallas import tpu as pltpu
from jax.experimental.pallas import tpu_sc as plsc

_N_NODES = 100
_N_STATES = 1000
_BATCH = 16384
_TSTRIDE = 1024
_TWORDS = _N_NODES * _TSTRIDE

_NW = 32
_SPW = _BATCH // _NW
_HC = 128
_NH = _SPW // _HC
_GRP = _HC // 16


def _prep_body(l_ref, tab_ref):
    l = l_ref[...]
    m = jnp.max(l, axis=1, keepdims=True)
    s = jnp.sum(jnp.exp(l - m), axis=1, keepdims=True)
    lse = jnp.log(s) + m
    t = (l - lse) * jnp.float32(1.0 / _N_NODES)
    tp = jnp.concatenate(
        [t, jnp.zeros((_N_NODES, _TSTRIDE - _N_STATES), jnp.float32)], axis=1)
    tab_ref[...] = tp.reshape(_TWORDS // 128, 128)


def _prep_table(logits):
    return pl.pallas_call(
        _prep_body,
        out_shape=jax.ShapeDtypeStruct((_TWORDS // 128, 128), jnp.float32),
    )(logits)


def _sc_gather_sum(xt, tab):
    mesh = plsc.VectorSubcoreMesh(core_axis_name="c", subcore_axis_name="s")

    @functools.partial(
        pl.kernel,
        mesh=mesh,
        out_type=jax.ShapeDtypeStruct((_BATCH,), jnp.float32),
        compiler_params=pltpu.CompilerParams(needs_layout_passes=False),
        scratch_types=[
            pltpu.VMEM((_TWORDS,), jnp.float32),
            pltpu.VMEM((_N_NODES, _HC), jnp.int32),
            pltpu.VMEM((_N_NODES, _HC), jnp.int32),
            pltpu.VMEM((_SPW,), jnp.float32),
            pltpu.SemaphoreType.DMA,
            pltpu.SemaphoreType.DMA,
            pltpu.SemaphoreType.DMA,
            pltpu.SemaphoreType.DMA,
        ],
    )
    def k(xt_hbm, tab_hbm, out_hbm,
          tab_v, xa_v, xb_v, out_v,
          sem_t, sem_xa, sem_xb, sem_o):
        wid = lax.axis_index("s") * 2 + lax.axis_index("c")
        base = wid * _SPW

        h_t = pltpu.async_copy(tab_hbm, tab_v, sem_t)
        xbufs = (xa_v, xb_v)
        xsems = (sem_xa, sem_xb)
        h = [None, None]
        h[0] = pltpu.async_copy(xt_hbm.at[:, pl.ds(base, _HC)], xa_v, sem_xa)
        h[1] = pltpu.async_copy(
            xt_hbm.at[:, pl.ds(base + _HC, _HC)], xb_v, sem_xb)
        h_t.wait()
        zero = jnp.zeros((16,), jnp.float32)

        def make_body(xv):
            def body(n, accs):
                noff = n * _TSTRIDE
                new = []
                for g in range(_GRP):
                    xrow = xv[n, pl.ds(g * 16, 16)]
                    val = plsc.load_gather(tab_v, [xrow + noff])
                    new.append(accs[g] + val)
                return tuple(new)
            return body

        out_handles = []
        for hc in range(_NH):
            h[hc % 2].wait()
            xv = xbufs[hc % 2]
            accs = lax.fori_loop(0, _N_NODES, make_body(xv),
                                 (zero,) * _GRP, unroll=2)
            if hc + 2 < _NH:
                h[hc % 2] = pltpu.async_copy(
                    xt_hbm.at[:, pl.ds(base + (hc + 2) * _HC, _HC)],
                    xbufs[hc % 2], xsems[hc % 2])
            for g in range(_GRP):
                out_v[pl.ds(hc * _HC + g * 16, 16)] = accs[g]
            out_handles.append(pltpu.async_copy(
                out_v.at[pl.ds(hc * _HC, _HC)],
                out_hbm.at[pl.ds(base + hc * _HC, _HC)], sem_o))
        for oh in out_handles:
            oh.wait()

    return k(xt, tab)


def kernel(x, logits):
    tab = _prep_table(logits)
    return _sc_gather_sum(x.T, tab.reshape(-1))
